# baseline (speedup 1.0000x reference)
import functools

import jax
import jax.numpy as jnp
from jax import lax
from jax.experimental import pallas as pl
from jax.experimental.pallas import tpu as pltpu
from jax.experimental.pallas import tpu_sc as plsc

_B, _N, _C = 16, 300, 92
_T = 50
_TP = 64
_NP = 304
_G = 262144
_G_BLK = 8192
_NBLK = _G // _G_BLK
_INF = float("inf")


def _global_loss_body(cls_p, cls_t, reg_p, reg_t, out_ref):
    i = pl.program_id(0)

    @pl.when(i == 0)
    def _():
        out_ref[0] = 0.0
        out_ref[1] = 0.0

    s_cls = jnp.sum(jnp.abs(cls_p[...] - cls_t[...]))
    d = reg_p[...] - reg_t[...]
    out_ref[0] += s_cls
    out_ref[1] += jnp.sum(d * d)


def _build_body(coords_ref, logits_ref, tgt_ref, labels_ref,
                cm_ref, cbb_ref, cgi_ref, rsb_ref):
    logits = logits_ref[...]
    rowsum = jnp.sum(logits, axis=2, keepdims=True)
    prob = jax.nn.softmax(logits, axis=-1)

    valid = jax.lax.broadcasted_iota(jnp.int32, (_N, _TP), 1) < _T

    for b in range(_B):
        prob_b = prob[b]
        lab = labels_ref[b]
        oh = (lab == jax.lax.broadcasted_iota(jnp.int32, (_C, _TP), 0))
        cclass = jax.lax.dot(prob_b, oh.astype(jnp.float32),
                             precision=jax.lax.Precision.HIGHEST)

        cb = coords_ref[b]
        cx, cy, w, h = cb[:, 0:1], cb[:, 1:2], cb[:, 2:3], cb[:, 3:4]
        tg = tgt_ref[b]
        tcx, tcy, tw, th = tg[0:1, :], tg[1:2, :], tg[2:3, :], tg[3:4, :]

        cbbox = (jnp.abs(cx - tcx) + jnp.abs(cy - tcy)
                 + jnp.abs(w - tw) + jnp.abs(h - th))

        x0, y0 = cx - 0.5 * w, cy - 0.5 * h
        x1, y1 = cx + 0.5 * w, cy + 0.5 * h
        tx0, ty0 = tcx - 0.5 * tw, tcy - 0.5 * th
        tx1, ty1 = tcx + 0.5 * tw, tcy + 0.5 * th
        area1 = (x1 - x0) * (y1 - y0)
        area2 = (tx1 - tx0) * (ty1 - ty0)
        inter = (jnp.clip(jnp.minimum(x1, tx1) - jnp.maximum(x0, tx0), 0.0)
                 * jnp.clip(jnp.minimum(y1, ty1) - jnp.maximum(y0, ty0), 0.0))
        union = area1 + area2 - inter
        iou = inter / union
        areae = (jnp.clip(jnp.maximum(x1, tx1) - jnp.minimum(x0, tx0), 0.0)
                 * jnp.clip(jnp.maximum(y1, ty1) - jnp.minimum(y0, ty0), 0.0))
        giou = iou - (areae - union) / areae

        cm = 5.0 * cbbox - cclass - 2.0 * giou
        cm_ref[b] = jnp.where(valid, cm, _INF)
        cbb_ref[b] = jnp.where(valid, cbbox, 0.0)
        cgi_ref[b] = jnp.where(valid, giou, 0.0)
        rsb_ref[b] = jnp.broadcast_to(rowsum[b], (_N, _TP))


def _sc_match_body(cm_hbm, cbb_hbm, cgi_hbm, rsb_hbm, out_hbm,
                   cm_v, cbb_v, cgi_v, rsb_v, rowmin_v, colpen_v, acc_v):
    wid = lax.axis_index("s") * 2 + lax.axis_index("c")
    lane = lax.iota(jnp.int32, 16)
    lane0 = lane == 0
    big = jnp.int32(2 ** 30)

    @pl.when(wid < _B)
    def _():
        b = wid
        pltpu.sync_copy(cm_hbm.at[b], cm_v)
        pltpu.sync_copy(cbb_hbm.at[b], cbb_v)
        pltpu.sync_copy(cgi_hbm.at[b], cgi_v)
        pltpu.sync_copy(rsb_hbm.at[b], rsb_v)

        for k in range(_TP // 16):
            colpen_v[pl.ds(k * 16, 16)] = jnp.where(
                k * 16 + lane < _T, 0.0, _INF)

        for k in range(_N // 16, _NP // 16):
            rowmin_v[pl.ds(k * 16, 16)] = jnp.full((16,), _INF)

        def init_row(r, _):
            rvec = jnp.full((16,), r, jnp.int32)
            m = jnp.full((16,), _INF)
            for k in range(_TP // 16):
                m = jnp.minimum(
                    m, plsc.load_gather(
                        cm_v, [rvec * _TP + k * 16 + lane]))
            plsc.store_scatter(rowmin_v, [rvec],
                               jnp.full((16,), jnp.min(m)), mask=lane0)
            return 0

        lax.fori_loop(0, _N, init_row, 0)

        def pick_row():
            best_v = jnp.full((16,), _INF)
            best_i = jnp.zeros((16,), jnp.int32)
            for k in range(_NP // 16):
                v = rowmin_v[pl.ds(k * 16, 16)]
                upd = v < best_v
                best_v = jnp.where(upd, v, best_v)
                best_i = jnp.where(upd, k * 16 + lane, best_i)
            gmin = jnp.min(best_v)
            return jnp.min(jnp.where(best_v == gmin, best_i, big))

        def rescan_row(i_s):
            ivec = jnp.full((16,), i_s, jnp.int32)
            bv = jnp.full((16,), _INF)
            bj = jnp.zeros((16,), jnp.int32)
            for k in range(_TP // 16):
                v = (plsc.load_gather(cm_v, [ivec * _TP + k * 16 + lane])
                     + colpen_v[pl.ds(k * 16, 16)])
                upd = v < bv
                bv = jnp.where(upd, v, bv)
                bj = jnp.where(upd, k * 16 + lane, bj)
            mrow = jnp.min(bv)
            j_s = jnp.min(jnp.where(bv == mrow, bj, big))
            return mrow, j_s

        def w_cond(c):
            return c[0] == 0

        def w_body(c):
            i_s = pick_row()
            mrow, j_s = rescan_row(i_s)
            ivec = jnp.full((16,), i_s, jnp.int32)
            cached = jnp.min(plsc.load_gather(rowmin_v, [ivec]))
            stale = mrow > cached

            @pl.when(stale)
            def _():
                plsc.store_scatter(rowmin_v, [ivec],
                                   jnp.full((16,), mrow), mask=lane0)

            ok = jnp.where(stale, 0, 1).astype(jnp.int32)
            return (ok, i_s, j_s)

        def step(t, accs):
            acc_bb, acc_gi, acc_rs = accs
            _, i_s, j_s = lax.while_loop(
                w_cond, w_body,
                (jnp.int32(0), jnp.int32(0), jnp.int32(0)))
            ivec = jnp.full((16,), i_s, jnp.int32)
            jvec = jnp.full((16,), j_s, jnp.int32)
            fvec = ivec * _TP + jvec
            acc_bb = acc_bb + plsc.load_gather(cbb_v, [fvec])
            acc_gi = acc_gi + plsc.load_gather(cgi_v, [fvec])
            acc_rs = acc_rs + plsc.load_gather(rsb_v, [fvec])
            plsc.store_scatter(rowmin_v, [ivec],
                               jnp.full((16,), _INF), mask=lane0)
            plsc.store_scatter(colpen_v, [jvec],
                               jnp.full((16,), _INF), mask=lane0)
            return (acc_bb, acc_gi, acc_rs)

        zero = jnp.zeros((16,), jnp.float32)
        acc_bb, acc_gi, acc_rs = lax.fori_loop(
            0, _T, step, (zero, zero, zero))

        acc_v[pl.ds(0, 16)] = acc_bb
        acc_v[pl.ds(16, 16)] = acc_gi
        acc_v[pl.ds(32, 16)] = acc_rs
        acc_v[pl.ds(48, 16)] = zero
        pltpu.sync_copy(acc_v, out_hbm.at[b])


def _sc_match(cm, cbb, cgi, rsb):
    mesh = plsc.VectorSubcoreMesh(core_axis_name="c", subcore_axis_name="s",
                                  num_cores=2, num_subcores=16)
    f = functools.partial(
        pl.kernel,
        out_type=jax.ShapeDtypeStruct((_B, 64), jnp.float32),
        mesh=mesh,
        compiler_params=pltpu.CompilerParams(needs_layout_passes=False),
        scratch_types=[
            pltpu.VMEM((_N * _TP,), jnp.float32),
            pltpu.VMEM((_N * _TP,), jnp.float32),
            pltpu.VMEM((_N * _TP,), jnp.float32),
            pltpu.VMEM((_N * _TP,), jnp.float32),
            pltpu.VMEM((_NP,), jnp.float32),
            pltpu.VMEM((_TP,), jnp.float32),
            pltpu.VMEM((64,), jnp.float32),
        ],
    )(_sc_match_body)
    return f(cm, cbb, cgi, rsb)


def kernel(box_coords, box_logits, tgt_boxes, g_cls_pred, g_cls_tgt,
           g_regr_pred, g_regr_tgt, tgt_labels):
    tgt_t = jnp.transpose(tgt_boxes, (0, 2, 1))
    tgt_t = jnp.pad(tgt_t, ((0, 0), (0, 0), (0, _TP - _T)))
    labels = jnp.pad(tgt_labels.astype(jnp.int32),
                     ((0, 0), (0, _TP - _T)),
                     constant_values=-1)[:, None, :]

    plane = jax.ShapeDtypeStruct((_B, _N, _TP), jnp.float32)
    cm, cbb, cgi, rsb = pl.pallas_call(
        _build_body,
        out_shape=[plane, plane, plane, plane],
    )(box_coords, box_logits, tgt_t, labels)

    gsums = pl.pallas_call(
        _global_loss_body,
        grid=(_NBLK,),
        in_specs=[pl.BlockSpec((_B, _G_BLK), lambda i: (0, i))] * 4,
        out_specs=pl.BlockSpec(memory_space=pltpu.SMEM),
        out_shape=jax.ShapeDtypeStruct((2,), jnp.float32),
    )(g_cls_pred, g_cls_tgt, g_regr_pred, g_regr_tgt)

    parts = _sc_match(cm.reshape(_B, _N * _TP), cbb.reshape(_B, _N * _TP),
                      cgi.reshape(_B, _N * _TP), rsb.reshape(_B, _N * _TP))
    msums = jnp.sum(parts[:, 0::16], axis=0)

    denom = jnp.float32(_B * _G)
    num_boxes = jnp.float32(4.0 * _B)
    g_cls_loss = gsums[0] / denom
    g_regr_loss = gsums[1] / denom
    loss_bbox = msums[0] / num_boxes
    loss_giou = (jnp.float32(_B * _T) - msums[1]) / num_boxes
    loss_cls = -msums[2]
    return jnp.stack([g_cls_loss, g_regr_loss, loss_bbox, loss_giou,
                      loss_cls])
